# baseline (speedup 1.0000x reference)
import jax
import jax.numpy as jnp
from jax import lax
from jax.experimental import pallas as pl
from jax.experimental.pallas import tpu as pltpu
from jax.experimental.pallas import tpu_sc as plsc

K = 16
D = 128

_L = 16
_NC = 2
_NW = 32
_RPW = D // _NW


def _sc_body(dens_ref, conf_ref, out_ref, buf, conf_v, sem):
    wid = lax.axis_index("s") * _NC + lax.axis_index("c")
    base = wid * _RPW
    pltpu.sync_copy(conf_ref, conf_v)
    conf = conf_v[...][0]
    row0 = conf * D + base
    gathers = [
        pltpu.async_copy(
            dens_ref.at[pl.ds(row0 + c, 1)], buf.at[c], sem.at[c]
        )
        for c in range(_RPW)
    ]
    for c in range(_RPW):
        gathers[c].wait()
        pltpu.sync_copy(buf.at[c], out_ref.at[pl.ds(base + c, 1)])


def kernel(density, conformation):
    dens3d = density.reshape(K * D, D, D)
    conf_vec = jnp.full((_L,), conformation, jnp.int32)
    mesh = plsc.VectorSubcoreMesh(core_axis_name="c", subcore_axis_name="s")
    sc_call = pl.kernel(
        _sc_body,
        out_type=jax.ShapeDtypeStruct((D, D, D), jnp.float32),
        mesh=mesh,
        scratch_types=[
            pltpu.VMEM((_RPW, 1, D, D), jnp.float32),
            pltpu.VMEM((_L,), jnp.int32),
            pltpu.SemaphoreType.DMA((_RPW,)),
        ],
    )
    return sc_call(dens3d, conf_vec)


# speedup vs baseline: 1.0339x; 1.0339x over previous
import jax
import jax.numpy as jnp
from jax import lax
from jax.experimental import pallas as pl
from jax.experimental.pallas import tpu as pltpu
from jax.experimental.pallas import tpu_sc as plsc

K = 16
D = 128

_L = 16
_NC = 2
_NW = 32
_RPW = D // _NW


def _sc_body(dens_ref, conf_ref, out_ref, buf, conf_v, sem):
    wid = lax.axis_index("s") * _NC + lax.axis_index("c")
    base = wid * _RPW
    pltpu.sync_copy(conf_ref, conf_v)
    conf = conf_v[...][0]
    src = dens_ref.at[pl.ds(conf * D + base, _RPW)]
    pltpu.async_copy(src, buf, sem).wait()
    pltpu.sync_copy(buf, out_ref.at[pl.ds(base, _RPW)])


def kernel(density, conformation):
    dens3d = density.reshape(K * D, D, D)
    conf_vec = jnp.full((_L,), conformation, jnp.int32)
    mesh = plsc.VectorSubcoreMesh(core_axis_name="c", subcore_axis_name="s")
    sc_call = pl.kernel(
        _sc_body,
        out_type=jax.ShapeDtypeStruct((D, D, D), jnp.float32),
        mesh=mesh,
        scratch_types=[
            pltpu.VMEM((_RPW, D, D), jnp.float32),
            pltpu.VMEM((_L,), jnp.int32),
            pltpu.SemaphoreType.DMA,
        ],
    )
    return sc_call(dens3d, conf_vec)
